# Initial kernel scaffold; baseline (speedup 1.0000x reference)
import jax
import jax.numpy as jnp
from jax.experimental import pallas as pl


def kernel(ufeat, ifeat, uhfeat, ihfeat, att0, basis0, Wdense0, Wh0, att1, basis1, Wdense1, Wh1, dec_W1, dec_W2, decCL_W1, decCL_W2, enc_edge_index, enc_edge_rating, dec_edge_index):
    raise NotImplementedError("write your pallas kernel here")


# speedup vs baseline: 15.9317x; 15.9317x over previous
import functools

import jax
import jax.numpy as jnp
from jax import lax
from jax.experimental import pallas as pl
from jax.experimental.pallas import tpu as pltpu
from jax.experimental.pallas import tpu_sc as plsc

NU = 5000
NI = 5000
D = 128
R = 5
E = 320000
ED = 100000
AGG = 128
OUT = 128

NC = 2
NS = 16
NW = NC * NS
CH = 128

VOFF = NU + 8
NACC = NU + 8 + NI + 104
RPT = NACC // NS
NP = 5120
RPTS = NP // NS

NCH_DEG = -(-2 * E // (NW * CH))
NCH_E = -(-E // (NW * CH))
NCH_D = -(-ED // (NW * CH))
EDP = NW * NCH_D * CH

_f32 = jnp.float32


def _mesh():
    return plsc.VectorSubcoreMesh(core_axis_name="c", subcore_axis_name="s",
                                  num_cores=NC, num_subcores=NS)


def _act(x):
    return jnp.where(x > 0, x, 0.1 * x)


def _pad_reshape(x, nslots, padval, nch):
    x = jnp.concatenate([x, jnp.full((nslots - x.shape[0],), padval, jnp.int32)])
    return x.reshape(NW, nch, CH)


def _sc_deg(idx):
    ones = jnp.ones((CH, 128), _f32)
    zeros = jnp.zeros((RPT, 128), _f32)

    @functools.partial(
        pl.kernel,
        out_type=jax.ShapeDtypeStruct((NC, NACC, 128), _f32),
        mesh=_mesh(),
        scratch_types=[
            pltpu.VMEM((NCH_DEG, CH), jnp.int32),
            pltpu.VMEM((CH, 128), _f32),
            pltpu.VMEM_SHARED((NACC, 128), _f32),
        ],
    )
    def body(idx_hbm, ones_hbm, zeros_hbm, out_hbm, idx_v, ones_v, acc_sh):
        cid = lax.axis_index("c")
        sid = lax.axis_index("s")
        wid = sid * NC + cid
        row0 = sid * RPT
        pltpu.sync_copy(idx_hbm.at[wid], idx_v)
        pltpu.sync_copy(ones_hbm, ones_v)
        pltpu.sync_copy(zeros_hbm, acc_sh.at[pl.ds(row0, RPT)])
        plsc.subcore_barrier()

        def chunk(ch, c):
            pltpu.sync_copy(ones_v, acc_sh.at[idx_v.at[ch]], add=True)
            return c

        lax.fori_loop(0, NCH_DEG, chunk, 0)
        plsc.subcore_barrier()
        pltpu.sync_copy(acc_sh.at[pl.ds(row0, RPT)],
                        out_hbm.at[cid, pl.ds(row0, RPT)])

    return body(idx, ones, zeros)


def _sc_side(tab, gi, si):
    zeros = jnp.zeros((RPTS, AGG), _f32)

    @functools.partial(
        pl.kernel,
        out_type=jax.ShapeDtypeStruct((NC, NP, AGG), _f32),
        mesh=_mesh(),
        scratch_types=[
            pltpu.VMEM((NCH_E, CH), jnp.int32),
            pltpu.VMEM((NCH_E, CH), jnp.int32),
            pltpu.VMEM((CH, AGG), _f32),
            pltpu.SemaphoreType.DMA,
            pltpu.VMEM_SHARED((NP, AGG), _f32),
        ],
    )
    def body(tab_hbm, gi_hbm, si_hbm, zeros_hbm, out_hbm,
             gi_v, si_v, rows_v, sem, acc_sh):
        cid = lax.axis_index("c")
        sid = lax.axis_index("s")
        wid = sid * NC + cid
        row0 = sid * RPTS
        pltpu.sync_copy(gi_hbm.at[wid], gi_v)
        pltpu.sync_copy(si_hbm.at[wid], si_v)
        pltpu.sync_copy(zeros_hbm, acc_sh.at[pl.ds(row0, RPTS)])
        plsc.subcore_barrier()

        def chunk(ch, c):
            pltpu.async_copy(tab_hbm.at[gi_v.at[ch]], rows_v, sem).wait()
            pltpu.sync_copy(rows_v, acc_sh.at[si_v.at[ch]], add=True)
            return c

        lax.fori_loop(0, NCH_E, chunk, 0)
        plsc.subcore_barrier()
        pltpu.sync_copy(acc_sh.at[pl.ds(row0, RPTS)],
                        out_hbm.at[cid, pl.ds(row0, RPTS)])

    return body(tab, gi, si, zeros)


def _sc_decgather(ut, vt, gdu, gdv):
    @functools.partial(
        pl.kernel,
        out_type=(jax.ShapeDtypeStruct((EDP, OUT), _f32),
                  jax.ShapeDtypeStruct((EDP, OUT), _f32)),
        mesh=_mesh(),
        scratch_types=[
            pltpu.VMEM((NCH_D, CH), jnp.int32),
            pltpu.VMEM((NCH_D, CH), jnp.int32),
            pltpu.VMEM((CH, OUT), _f32),
            pltpu.SemaphoreType.DMA,
        ],
    )
    def body(ut_hbm, vt_hbm, gdu_hbm, gdv_hbm, hu_hbm, hv_hbm,
             gdu_v, gdv_v, rows_v, sem):
        cid = lax.axis_index("c")
        sid = lax.axis_index("s")
        wid = sid * NC + cid
        base = wid * NCH_D * CH
        pltpu.sync_copy(gdu_hbm.at[wid], gdu_v)
        pltpu.sync_copy(gdv_hbm.at[wid], gdv_v)

        def chunk(ch, c):
            off = base + ch * CH
            pltpu.async_copy(ut_hbm.at[gdu_v.at[ch]], rows_v, sem).wait()
            pltpu.sync_copy(rows_v, hu_hbm.at[pl.ds(off, CH)])
            pltpu.async_copy(vt_hbm.at[gdv_v.at[ch]], rows_v, sem).wait()
            pltpu.sync_copy(rows_v, hv_hbm.at[pl.ds(off, CH)])
            return c

        lax.fori_loop(0, NCH_D, chunk, 0)

    return body(ut, vt, gdu, gdv)


def _k_prep(degp):
    def body(degp_ref, out_ref):
        d = degp_ref[0, :, 0:1] + degp_ref[1, :, 0:1]
        out_ref[...] = lax.rsqrt(d + 1.0)

    return pl.pallas_call(
        body, out_shape=jax.ShapeDtypeStruct((NACC, 1), _f32))(degp)


_BN = 1000


def _k_transform(x, scale, att, basis3):
    n = x.shape[0]
    grid = (R, n // _BN)

    def body(att_ref, x_ref, scale_ref, basis_ref, out_ref):
        r = pl.program_id(0)
        w = att_ref[r, 0] * basis_ref[0] + att_ref[r, 1] * basis_ref[1]
        xs = x_ref[...] * scale_ref[...]
        out_ref[...] = jnp.dot(xs, w, preferred_element_type=_f32)

    return pl.pallas_call(
        body,
        grid=grid,
        in_specs=[
            pl.BlockSpec(memory_space=pltpu.SMEM),
            pl.BlockSpec((_BN, D), lambda r, i: (i, 0)),
            pl.BlockSpec((_BN, 1), lambda r, i: (i, 0)),
            pl.BlockSpec((2, D, AGG), lambda r, i: (0, 0, 0)),
        ],
        out_specs=pl.BlockSpec((_BN, AGG), lambda r, i: (r * (n // _BN) + i, 0)),
        out_shape=jax.ShapeDtypeStruct((R * n, AGG), _f32),
    )(att, x, scale, basis3)


def _k_post(praw, scale, Wdense, hf, Wh, prev_o, prev_h, prev_f, wgt):
    n = scale.shape[0]
    grid = (n // _BN,)

    def body(praw_ref, scale_ref, wd_ref, hf_ref, wh_ref, po_ref, ph_ref, pf_ref,
             oa_ref, ha_ref, fn_ref, fa_ref, s2_ref):
        agg = (praw_ref[0] + praw_ref[1]) * scale_ref[...]
        o = jnp.dot(_act(agg), wd_ref[...], preferred_element_type=_f32)
        h = _act(jnp.dot(hf_ref[...], wh_ref[...], preferred_element_type=_f32))
        fn = o + h
        oa_ref[...] = po_ref[...] + wgt * o
        ha_ref[...] = ph_ref[...] + wgt * h
        fn_ref[...] = fn
        fa_ref[...] = pf_ref[...] + wgt * fn

        @pl.when(pl.program_id(0) == 0)
        def _():
            s2_ref[0, 0] = 0.0

        s2_ref[0, 0] += jnp.sum(h * h)

    blk = lambda i: (i, 0)
    return pl.pallas_call(
        body,
        grid=grid,
        in_specs=[
            pl.BlockSpec((2, _BN, AGG), lambda i: (0, i, 0)),
            pl.BlockSpec((_BN, 1), blk),
            pl.BlockSpec((AGG, OUT), lambda i: (0, 0)),
            pl.BlockSpec((_BN, D), blk),
            pl.BlockSpec((D, OUT), lambda i: (0, 0)),
            pl.BlockSpec((_BN, OUT), blk),
            pl.BlockSpec((_BN, OUT), blk),
            pl.BlockSpec((_BN, OUT), blk),
        ],
        out_specs=[
            pl.BlockSpec((_BN, OUT), blk),
            pl.BlockSpec((_BN, OUT), blk),
            pl.BlockSpec((_BN, OUT), blk),
            pl.BlockSpec((_BN, OUT), blk),
            pl.BlockSpec(memory_space=pltpu.SMEM),
        ],
        out_shape=[
            jax.ShapeDtypeStruct((n, OUT), _f32),
            jax.ShapeDtypeStruct((n, OUT), _f32),
            jax.ShapeDtypeStruct((n, OUT), _f32),
            jax.ShapeDtypeStruct((n, OUT), _f32),
            jax.ShapeDtypeStruct((1, 1), _f32),
        ],
    )(praw, scale, Wdense, hf, Wh, prev_o, prev_h, prev_f)


_BQ = 200


def _k_nce(q, k):
    n = q.shape[0]
    grid = (n // _BQ,)

    def body(q_ref, k_ref, kd_ref, out_ref):
        i = pl.program_id(0)
        qv = q_ref[...]
        qn = qv / (jnp.sqrt(jnp.sum(qv * qv, axis=1, keepdims=True)) + 1e-8)
        kv = k_ref[...]
        kn = kv / (jnp.sqrt(jnp.sum(kv * kv, axis=1, keepdims=True)) + 1e-8)
        logits = lax.dot_general(qn, kn, (((1,), (1,)), ((), ())),
                                 preferred_element_type=_f32) * 10.0
        m = jnp.max(logits, axis=1, keepdims=True)
        lse = m[:, 0] + jnp.log(jnp.sum(jnp.exp(logits - m), axis=1))
        kd = kd_ref[...]
        kdn = kd / (jnp.sqrt(jnp.sum(kd * kd, axis=1, keepdims=True)) + 1e-8)
        pos = jnp.sum(qn * kdn, axis=1) * 10.0

        @pl.when(i == 0)
        def _():
            out_ref[0, 0] = 0.0

        out_ref[0, 0] += jnp.sum(lse - pos)

    return pl.pallas_call(
        body,
        grid=grid,
        in_specs=[
            pl.BlockSpec((_BQ, OUT), lambda i: (i, 0)),
            pl.BlockSpec((n, OUT), lambda i: (0, 0)),
            pl.BlockSpec((_BQ, OUT), lambda i: (i, 0)),
        ],
        out_specs=pl.BlockSpec(memory_space=pltpu.SMEM),
        out_shape=jax.ShapeDtypeStruct((1, 1), _f32),
    )(q, k, k)


_BE = 2048


def _k_decoder(hu, hv, W1, W2, W1c, W2c):
    grid = (EDP // _BE,)

    def body(hu_ref, hv_ref, w1_ref, w2_ref, w1c_ref, w2c_ref, p_ref, pc_ref):
        u = hu_ref[...]
        v = hv_ref[...]
        t = jnp.maximum(
            jnp.dot(u, w1_ref[0:OUT, :], preferred_element_type=_f32)
            + jnp.dot(v, w1_ref[OUT:2 * OUT, :], preferred_element_type=_f32), 0.0)
        p_ref[...] = jnp.dot(t, w2_ref[...], preferred_element_type=_f32)
        tc = jnp.maximum(
            jnp.dot(u, w1c_ref[0:OUT, :], preferred_element_type=_f32)
            + jnp.dot(v, w1c_ref[OUT:2 * OUT, :], preferred_element_type=_f32), 0.0)
        pc_ref[...] = jnp.dot(tc, w2c_ref[...], preferred_element_type=_f32)

    blk = lambda i: (i, 0)
    full = lambda i: (0, 0)
    return pl.pallas_call(
        body,
        grid=grid,
        in_specs=[
            pl.BlockSpec((_BE, OUT), blk),
            pl.BlockSpec((_BE, OUT), blk),
            pl.BlockSpec((2 * OUT, 128), full),
            pl.BlockSpec((128, R), full),
            pl.BlockSpec((2 * OUT, 128), full),
            pl.BlockSpec((128, R), full),
        ],
        out_specs=[pl.BlockSpec((_BE, R), blk), pl.BlockSpec((_BE, R), blk)],
        out_shape=[jax.ShapeDtypeStruct((EDP, R), _f32),
                   jax.ShapeDtypeStruct((EDP, R), _f32)],
    )(hu, hv, W1, W2, W1c, W2c)


def _k_reg(att, basis3):
    def body(att_ref, basis_ref, out_ref):
        ws = [att_ref[r, 0] * basis_ref[0] + att_ref[r, 1] * basis_ref[1]
              for r in range(R)]
        acc = jnp.zeros((), _f32)
        for i in range(1, R):
            num = jnp.sum(ws[i] * ws[i - 1], axis=1)
            den = (jnp.sqrt(jnp.sum(ws[i] * ws[i], axis=1))
                   * jnp.sqrt(jnp.sum(ws[i - 1] * ws[i - 1], axis=1)) + 1e-8)
            acc = acc - jnp.sum(num / den)
        out_ref[0, 0] = acc

    return pl.pallas_call(
        body,
        in_specs=[
            pl.BlockSpec(memory_space=pltpu.SMEM),
            pl.BlockSpec((2, D, AGG), lambda: (0, 0, 0)),
        ],
        out_specs=pl.BlockSpec(memory_space=pltpu.SMEM),
        out_shape=jax.ShapeDtypeStruct((1, 1), _f32),
    )(att, basis3)


def kernel(ufeat, ifeat, uhfeat, ihfeat, att0, basis0, Wdense0, Wh0, att1,
           basis1, Wdense1, Wh1, dec_W1, dec_W2, decCL_W1, decCL_W2,
           enc_edge_index, enc_edge_rating, dec_edge_index):
    src = enc_edge_index[0].astype(jnp.int32)
    dst = enc_edge_index[1].astype(jnp.int32)
    rat = enc_edge_rating.astype(jnp.int32)
    dsrc = dec_edge_index[0].astype(jnp.int32)
    ddst = dec_edge_index[1].astype(jnp.int32)

    basis0_3 = basis0.reshape(2, D, AGG)
    basis1_3 = basis1.reshape(2, D, AGG)

    idx_deg = _pad_reshape(jnp.concatenate([src, VOFF + dst]),
                           NW * NCH_DEG * CH, NU, NCH_DEG)
    gi_u = _pad_reshape(rat * NI + dst, NW * NCH_E * CH, 0, NCH_E)
    si_u = _pad_reshape(src, NW * NCH_E * CH, NU, NCH_E)
    gi_i = _pad_reshape(rat * NU + src, NW * NCH_E * CH, 0, NCH_E)
    si_i = _pad_reshape(dst, NW * NCH_E * CH, NI, NCH_E)
    gdu = _pad_reshape(dsrc, EDP, 0, NCH_D)
    gdv = _pad_reshape(ddst, EDP, 0, NCH_D)

    degp = _sc_deg(idx_deg)
    scale = _k_prep(degp)
    a = scale[:NU]
    b = scale[VOFF:VOFF + NI]

    zeros_n = jnp.zeros((NU, OUT), _f32)

    tab_u0 = _k_transform(ufeat, a, att0, basis0_3)
    tab_v0 = _k_transform(ifeat, b, att0, basis0_3)
    accu0 = _sc_side(tab_v0, gi_u, si_u)
    acci0 = _sc_side(tab_u0, gi_i, si_i)
    uo0, uh0, uf1, _, s2u0 = _k_post(accu0[:, :NU], a, Wdense0, uhfeat, Wh0,
                                     zeros_n, zeros_n, zeros_n, 1.0)
    mo0, mh0, vf1, _, s2m0 = _k_post(acci0[:, :NI], b, Wdense0,
                                     ihfeat, Wh0, zeros_n, zeros_n, zeros_n, 1.0)
    nce0u = _k_nce(uo0, uh0)
    nce0m = _k_nce(mo0, mh0)

    tab_u1 = _k_transform(uf1, a, att1, basis1_3)
    tab_v1 = _k_transform(vf1, b, att1, basis1_3)
    accu1 = _sc_side(tab_v1, gi_u, si_u)
    acci1 = _sc_side(tab_u1, gi_i, si_i)
    ul, uhyp, _, uout, s2u1 = _k_post(accu1[:, :NU], a, Wdense1, uhfeat, Wh1,
                                      uo0, uh0, uf1, 0.5)
    ml, mhyp, _, mout, s2m1 = _k_post(acci1[:, :NI], b, Wdense1,
                                      ihfeat, Wh1, mo0, mh0, vf1, 0.5)
    nce1u = _k_nce(ul, uhyp)
    nce1m = _k_nce(ml, mhyp)

    closs = (nce0u[0, 0] + nce0m[0, 0] + nce1u[0, 0] + nce1m[0, 0]) / float(NU)
    rcloss = (s2u0[0, 0] + s2m0[0, 0] + s2u1[0, 0] + s2m1[0, 0]) / float(NU * OUT)

    hu, hv = _sc_decgather(uout, mout, gdu, gdv)
    p, pc = _k_decoder(hu, hv, dec_W1, dec_W2, decCL_W1, decCL_W2)
    pred_ratings = p[:ED]
    predcl = pc[:ED]

    reg_loss = _k_reg(att0, basis0_3)[0, 0]
    return pred_ratings, reg_loss, closs, rcloss, predcl
